# baseline (speedup 1.0000x reference)
import jax
import jax.numpy as jnp
from jax import lax
from jax.experimental import pallas as pl
from jax.experimental.pallas import tpu as pltpu
from jax.experimental.pallas import tpu_sc as plsc

_ROW_BLOCK = 1024
_TOKENS = 8192
_KEXP = 64
_NC, _NS = 2, 16
_NW = _NC * _NS
_RPW = _TOKENS // _NW
_BLK = 16
_NSTREAM = 4
_STRIDE = _KEXP + 1


def _logits_body(z_ref, wt_ref, b_ref, out_ref):
    acc = jnp.dot(z_ref[...], wt_ref[...], preferred_element_type=jnp.float32)
    out_ref[...] = acc + b_ref[0:1, :]


@jax.jit
def _logits(z, wt, b2d):
    tokens, dim = z.shape
    kexp = wt.shape[1]
    return pl.pallas_call(
        _logits_body,
        grid=(tokens // _ROW_BLOCK,),
        in_specs=[
            pl.BlockSpec((_ROW_BLOCK, dim), lambda i: (i, 0)),
            pl.BlockSpec((dim, kexp), lambda i: (0, 0)),
            pl.BlockSpec((8, kexp), lambda i: (0, 0)),
        ],
        out_specs=pl.BlockSpec((_ROW_BLOCK, kexp), lambda i: (i, 0)),
        out_shape=jax.ShapeDtypeStruct((tokens, kexp), jnp.float32),
    )(z, wt, b2d)


def _merge_top2(a, b):
    am1, ai1, am2, ai2 = a
    bm1, bi1, bm2, bi2 = b
    b_wins = bm1 > am1
    m1 = jnp.where(b_wins, bm1, am1)
    i1 = jnp.where(b_wins, bi1, ai1)
    cand_v = jnp.where(b_wins, am1, bm1)
    cand_i = jnp.where(b_wins, ai1, bi1)
    alt_v = jnp.where(b_wins, bm2, am2)
    alt_i = jnp.where(b_wins, bi2, ai2)
    alt_better = jnp.where(b_wins, alt_v > cand_v, alt_v >= cand_v)
    m2 = jnp.where(alt_better, alt_v, cand_v)
    i2 = jnp.where(alt_better, alt_i, cand_i)
    return m1, i1, m2, i2


def _sc_router_body(logits_hbm, out_hbm, stage_v, lg_v, out_v):
    wid = lax.axis_index("s") * _NC + lax.axis_index("c")
    base_row = wid * _RPW
    pltpu.sync_copy(logits_hbm.at[pl.ds(base_row, _RPW)], stage_v)

    lane = lax.iota(jnp.int32, _BLK)
    lane65 = lane * _STRIDE
    zeros16 = jnp.zeros((_BLK,), jnp.float32)
    epg = _KEXP // _NSTREAM

    def block(b, carry):
        rows = b * _BLK + lane
        for r in range(_BLK):
            for c in range(_KEXP // _BLK):
                lg_v[pl.ds((b * _BLK + r) * _STRIDE + c * _BLK, _BLK)] = (
                    stage_v[b * _BLK + r, pl.ds(c * _BLK, _BLK)])
        neg = jnp.full((_BLK,), -3e38, jnp.float32)
        zi = jnp.zeros((_BLK,), jnp.int32)
        chains = []
        for s in range(_NSTREAM):
            chains.append([neg, zi, neg, zi])
        for j in range(epg):
            for s in range(_NSTREAM):
                e = s * epg + j
                e_v = jnp.full((_BLK,), e, jnp.int32)
                v = plsc.load_gather(lg_v, [b * (_BLK * _STRIDE) + lane65 + e])
                m1, i1, m2, i2 = chains[s]
                gt1 = v > m1
                gt2 = v > m2
                chains[s] = [
                    jnp.where(gt1, v, m1),
                    jnp.where(gt1, e_v, i1),
                    jnp.where(gt1, m1, jnp.where(gt2, v, m2)),
                    jnp.where(gt1, i1, jnp.where(gt2, e_v, i2)),
                ]
        t01 = _merge_top2(tuple(chains[0]), tuple(chains[1]))
        t23 = _merge_top2(tuple(chains[2]), tuple(chains[3]))
        m1, i1, m2, i2 = _merge_top2(t01, t23)
        t = jnp.exp(m2 - m1)
        den = 1.0 + t
        w1 = 1.0 / den
        w2 = t / den
        for r in range(_BLK):
            for c in range(_KEXP // _BLK):
                out_v[b * _BLK + r, pl.ds(c * _BLK, _BLK)] = zeros16
        plsc.store_scatter(out_v, [rows, i1], w1)
        plsc.store_scatter(out_v, [rows, i2], w2)
        return carry

    lax.fori_loop(0, _RPW // _BLK, block, 0)
    pltpu.sync_copy(out_v, out_hbm.at[pl.ds(base_row, _RPW)])


@jax.jit
def _sc_router(logits):
    mesh = plsc.VectorSubcoreMesh(
        core_axis_name="c", subcore_axis_name="s",
        num_cores=_NC, num_subcores=_NS,
    )
    return pl.kernel(
        _sc_router_body,
        out_type=jax.ShapeDtypeStruct((_TOKENS, _KEXP), jnp.float32),
        mesh=mesh,
        scratch_types=[
            pltpu.VMEM((_RPW, _KEXP), jnp.float32),
            pltpu.VMEM((_RPW * _STRIDE,), jnp.float32),
            pltpu.VMEM((_RPW, _KEXP), jnp.float32),
        ],
        compiler_params=pltpu.CompilerParams(
            needs_layout_passes=False,
            use_tc_tiling_on_sc=True,
            skip_device_barrier=True,
        ),
    )(logits)


def kernel(z, W, b, k):
    del k
    wt = W.T
    b2d = jnp.broadcast_to(b[None, :], (8, b.shape[0]))
    return _sc_router(_logits(z, wt, b2d))


# speedup vs baseline: 1.0008x; 1.0008x over previous
import jax
import jax.numpy as jnp
from jax import lax
from jax.experimental import pallas as pl
from jax.experimental.pallas import tpu as pltpu
from jax.experimental.pallas import tpu_sc as plsc

_ROW_BLOCK = 1024
_TOKENS = 8192
_KEXP = 64
_NC, _NS = 2, 16
_NW = _NC * _NS
_RPW = _TOKENS // _NW
_BLK = 16
_NSTREAM = 4
_STRIDE = _KEXP + 1


def _logits_body(z_ref, wt_ref, b_ref, out_ref):
    acc = jnp.dot(z_ref[...], wt_ref[...], preferred_element_type=jnp.float32)
    out_ref[...] = acc + b_ref[0:1, :]


@jax.jit
def _logits(z, wt, b2d):
    tokens, dim = z.shape
    kexp = wt.shape[1]
    return pl.pallas_call(
        _logits_body,
        grid=(tokens // _ROW_BLOCK,),
        in_specs=[
            pl.BlockSpec((_ROW_BLOCK, dim), lambda i: (i, 0)),
            pl.BlockSpec((dim, kexp), lambda i: (0, 0)),
            pl.BlockSpec((8, kexp), lambda i: (0, 0)),
        ],
        out_specs=pl.BlockSpec((_ROW_BLOCK, kexp), lambda i: (i, 0)),
        out_shape=jax.ShapeDtypeStruct((tokens, kexp), jnp.float32),
    )(z, wt, b2d)


def _merge_top2(a, b):
    am1, ai1, am2, ai2 = a
    bm1, bi1, bm2, bi2 = b
    b_wins = bm1 > am1
    m1 = jnp.where(b_wins, bm1, am1)
    i1 = jnp.where(b_wins, bi1, ai1)
    cand_v = jnp.where(b_wins, am1, bm1)
    cand_i = jnp.where(b_wins, ai1, bi1)
    alt_v = jnp.where(b_wins, bm2, am2)
    alt_i = jnp.where(b_wins, bi2, ai2)
    alt_better = jnp.where(b_wins, alt_v > cand_v, alt_v >= cand_v)
    m2 = jnp.where(alt_better, alt_v, cand_v)
    i2 = jnp.where(alt_better, alt_i, cand_i)
    return m1, i1, m2, i2


def _sc_router_body(logits_hbm, out_hbm, stage_v, lg_v, out_v):
    wid = lax.axis_index("s") * _NC + lax.axis_index("c")
    base_row = wid * _RPW
    pltpu.sync_copy(logits_hbm.at[pl.ds(base_row, _RPW)], stage_v)

    lane = lax.iota(jnp.int32, _BLK)
    lane65 = lane * _STRIDE
    zeros16 = jnp.zeros((_BLK,), jnp.float32)
    epg = _KEXP // _NSTREAM

    def block(b, carry):
        rows = b * _BLK + lane
        for r in range(_BLK):
            for c in range(_KEXP // _BLK):
                lg_v[pl.ds((b * _BLK + r) * _STRIDE + c * _BLK, _BLK)] = (
                    stage_v[b * _BLK + r, pl.ds(c * _BLK, _BLK)])
        neg = jnp.full((_BLK,), -3e38, jnp.float32)
        zi = jnp.zeros((_BLK,), jnp.int32)
        chains = []
        for s in range(_NSTREAM):
            chains.append([neg, zi, neg, zi])
        for j in range(epg):
            for s in range(_NSTREAM):
                e = s * epg + j
                e_v = jnp.full((_BLK,), e, jnp.int32)
                v = plsc.load_gather(lg_v, [b * (_BLK * _STRIDE) + lane65 + e])
                m1, i1, m2, i2 = chains[s]
                gt1 = v > m1
                gt2 = v > m2
                chains[s] = [
                    jnp.where(gt1, v, m1),
                    jnp.where(gt1, e_v, i1),
                    jnp.where(gt1, m1, jnp.where(gt2, v, m2)),
                    jnp.where(gt1, i1, jnp.where(gt2, e_v, i2)),
                ]
        t01 = _merge_top2(tuple(chains[0]), tuple(chains[1]))
        t23 = _merge_top2(tuple(chains[2]), tuple(chains[3]))
        m1, i1, m2, i2 = _merge_top2(t01, t23)
        t = jnp.exp(m2 - m1)
        den = 1.0 + t
        w1 = 1.0 / den
        w2 = t / den
        for r in range(_BLK):
            for c in range(_KEXP // _BLK):
                out_v[b * _BLK + r, pl.ds(c * _BLK, _BLK)] = zeros16
        plsc.store_scatter(out_v, [rows, i1], w1)
        plsc.store_scatter(out_v, [rows, i2], w2)
        return carry

    lax.fori_loop(0, _RPW // _BLK, block, 0)
    pltpu.sync_copy(out_v, out_hbm.at[pl.ds(base_row, _RPW)])


@jax.jit
def _sc_router(logits):
    mesh = plsc.VectorSubcoreMesh(
        core_axis_name="c", subcore_axis_name="s",
        num_cores=_NC, num_subcores=_NS,
    )
    return pl.kernel(
        _sc_router_body,
        out_type=jax.ShapeDtypeStruct((_TOKENS, _KEXP), jnp.float32),
        mesh=mesh,
        scratch_types=[
            pltpu.VMEM((_RPW, _KEXP), jnp.float32),
            pltpu.VMEM((_RPW * _STRIDE,), jnp.float32),
            pltpu.VMEM((_RPW, _KEXP), jnp.float32),
        ],
        compiler_params=pltpu.CompilerParams(
            needs_layout_passes=False,
            use_tc_tiling_on_sc=True,
        ),
    )(logits)


def kernel(z, W, b, k):
    del k
    wt = W.T
    b2d = jnp.broadcast_to(b[None, :], (8, b.shape[0]))
    return _sc_router(_logits(z, wt, b2d))
